# baseline (speedup 1.0000x reference)
import functools

import jax
import jax.numpy as jnp
from jax import lax
from jax.experimental import pallas as pl
from jax.experimental.pallas import tpu as pltpu
from jax.experimental.pallas import tpu_sc as plsc

B = 4096
ROW = 20 * 64
NUM_ROWS = 100000
L = 16
NC, NS = 2, 16
NW = NC * NS
BPW = B // NW
C = 32
NCHUNK = BPW // C


def _build():
    mesh = plsc.VectorSubcoreMesh(core_axis_name="c", subcore_axis_name="s")

    @functools.partial(
        pl.kernel,
        mesh=mesh,
        out_type=jax.ShapeDtypeStruct((B, ROW), jnp.float32),
        scratch_types=[
            pltpu.VMEM((BPW,), jnp.int32),
            pltpu.VMEM((C, ROW), jnp.float32),
            pltpu.VMEM((C, ROW), jnp.float32),
            pltpu.SemaphoreType.DMA,
        ],
    )
    def run(x_hbm, idx_hbm, tab_hbm, out_hbm, idx_v, rows_v, xv, sem):
        wid = lax.axis_index("s") * NC + lax.axis_index("c")
        base = wid * BPW
        pltpu.sync_copy(idx_hbm.at[pl.ds(base, BPW)], idx_v)

        def chunk_body(c, carry):
            cb = base + c * C
            gather = pltpu.async_copy(
                tab_hbm.at[idx_v.at[pl.ds(c * C, C)]], rows_v, sem
            )
            pltpu.sync_copy(x_hbm.at[pl.ds(cb, C)], xv)
            gather.wait()

            def add_row(i, carry2):
                for j in range(ROW // L):
                    s = pl.ds(j * L, L)
                    plsc.addupdate(xv.at[i, s], rows_v[i, s])
                return carry2

            lax.fori_loop(0, C, add_row, 0)
            pltpu.sync_copy(xv, out_hbm.at[pl.ds(cb, C)])
            return carry

        lax.fori_loop(0, NCHUNK, chunk_body, 0)

    return run


_sc_call = _build()


@jax.jit
def kernel(x, idx, prompts):
    xf = x.reshape(B, ROW)
    tab = prompts.reshape(NUM_ROWS, ROW)
    out = _sc_call(xf, idx.astype(jnp.int32), tab)
    return out.reshape(x.shape)
